# baseline (speedup 1.0000x reference)
import functools

import jax
import jax.numpy as jnp
from jax import lax
from jax.experimental import pallas as pl
from jax.experimental.pallas import tpu as pltpu
from jax.experimental.pallas import tpu_sc as plsc

VOCAB = 100000
EMBED_DIM = 300
WIDTH_A = 256
WIDTH_B = 128
BATCH = 1024
SENT_LEN = 50
SENT_PAD = 56

NUM_CORES = 2
NUM_SUBCORES = 16
NUM_WORKERS = NUM_CORES * NUM_SUBCORES
ROWS_PER_WORKER = BATCH // NUM_WORKERS

VB = 2048
NVB = (VOCAB + VB - 1) // VB
IB = 128

_mesh = plsc.VectorSubcoreMesh(core_axis_name="c", subcore_axis_name="s")


def _t_in_a(wvt_ref, out_ref):
    out_ref[...] = wvt_ref[...].T


_transpose_in_a = pl.pallas_call(
    _t_in_a,
    grid=(NVB,),
    in_specs=[pl.BlockSpec((WIDTH_A, VB), lambda i: (0, i))],
    out_specs=pl.BlockSpec((VB, WIDTH_A), lambda i: (i, 0)),
    out_shape=jax.ShapeDtypeStruct((NVB * VB, WIDTH_A), jnp.float32),
)


def _t_in_b(wvt_ref, out_ref):
    out_ref[:, :64] = wvt_ref[...].T


_transpose_in_b = pl.pallas_call(
    _t_in_b,
    grid=(NVB,),
    in_specs=[pl.BlockSpec((64, VB), lambda i: (4, i))],
    out_specs=pl.BlockSpec((VB, WIDTH_B), lambda i: (i, 0)),
    out_shape=jax.ShapeDtypeStruct((NVB * VB, WIDTH_B), jnp.float32),
)


def _make_gather(width):
    @functools.partial(
        pl.kernel,
        mesh=_mesh,
        out_type=jax.ShapeDtypeStruct((BATCH, SENT_PAD, width), jnp.float32),
        scratch_types=[
            pltpu.VMEM((ROWS_PER_WORKER, SENT_PAD), jnp.int32),
            pltpu.VMEM((4, SENT_PAD, width), jnp.float32),
            pltpu.SemaphoreType.DMA,
            pltpu.SemaphoreType.DMA,
        ],
    )
    def _gather(idx_hbm, table_hbm, out_hbm, idx_v, rows_v, sem_g, sem_w):
        wid = lax.axis_index("s") * NUM_CORES + lax.axis_index("c")
        base = wid * ROWS_PER_WORKER
        pltpu.sync_copy(idx_hbm.at[wid], idx_v)

        pltpu.async_copy(table_hbm.at[idx_v.at[0]], rows_v.at[0], sem_g)
        pltpu.async_copy(table_hbm.at[idx_v.at[1]], rows_v.at[1], sem_g)

        def body(b, carry):
            pltpu.make_async_copy(out_hbm.at[0], rows_v.at[0], sem_g).wait()

            @pl.when(b >= 2)
            def _():
                pltpu.make_async_copy(out_hbm.at[0], rows_v.at[0],
                                      sem_w).wait()

            @pl.when(b + 2 < ROWS_PER_WORKER)
            def _():
                nxt = lax.rem(b + 2, 4)
                pltpu.async_copy(table_hbm.at[idx_v.at[b + 2]],
                                 rows_v.at[nxt], sem_g)

            cur = lax.rem(b, 4)
            pltpu.async_copy(rows_v.at[cur], out_hbm.at[base + b], sem_w)
            return carry

        lax.fori_loop(0, ROWS_PER_WORKER, body, 0, unroll=False)
        pltpu.make_async_copy(out_hbm.at[0], rows_v.at[0], sem_w).wait()
        pltpu.make_async_copy(out_hbm.at[0], rows_v.at[0], sem_w).wait()

    return _gather


_gather_a = _make_gather(WIDTH_A)
_gather_b = _make_gather(WIDTH_B)


def _t_out_kernel(ra_ref, rb_ref, out_ref):
    for j in range(SENT_LEN):
        ta = ra_ref[:, j, :].T
        tb = rb_ref[:, j, :].T[:EMBED_DIM - WIDTH_A]
        out_ref[j] = jnp.concatenate([ta, tb], axis=0)


_transpose_out = pl.pallas_call(
    _t_out_kernel,
    grid=(BATCH // IB,),
    in_specs=[
        pl.BlockSpec((IB, SENT_PAD, WIDTH_A), lambda i: (i, 0, 0)),
        pl.BlockSpec((IB, SENT_PAD, WIDTH_B), lambda i: (i, 0, 0)),
    ],
    out_specs=pl.BlockSpec((SENT_LEN, EMBED_DIM, IB), lambda i: (0, 0, i)),
    out_shape=jax.ShapeDtypeStruct((SENT_LEN, EMBED_DIM, BATCH), jnp.float32),
)


def kernel(x, word_vectors):
    wvt = jnp.swapaxes(word_vectors, 0, 1)
    table_a = _transpose_in_a(wvt)
    table_b = _transpose_in_b(wvt)
    idx3 = x.reshape(NUM_WORKERS, ROWS_PER_WORKER, SENT_LEN).astype(jnp.int32)
    fill = (
        jnp.arange(SENT_PAD - SENT_LEN, dtype=jnp.int32)[None, None, :]
        + 8 * jnp.arange(ROWS_PER_WORKER, dtype=jnp.int32)[None, :, None]
        + 256 * jnp.arange(NUM_WORKERS, dtype=jnp.int32)[:, None, None]
    )
    idxp = jnp.concatenate([idx3, fill], axis=2)
    rows_a = _gather_a(idxp, table_a)
    rows_b = _gather_b(idxp, table_b)
    g = _transpose_out(rows_a, rows_b)
    return jnp.transpose(g, (2, 0, 1))


# speedup vs baseline: 1.0749x; 1.0749x over previous
import functools

import jax
import jax.numpy as jnp
from jax import lax
from jax.experimental import pallas as pl
from jax.experimental.pallas import tpu as pltpu
from jax.experimental.pallas import tpu_sc as plsc

VOCAB = 100000
EMBED_DIM = 300
PAD_DIM = 384
BATCH = 1024
SENT_LEN = 50
SENT_PAD = 56

NUM_CORES = 2
NUM_SUBCORES = 16
NUM_WORKERS = NUM_CORES * NUM_SUBCORES
HALF = BATCH // 2
ROWS_PER_WORKER = HALF // NUM_WORKERS

VB = 2048
NVB = (VOCAB + VB - 1) // VB
IB = 128

_mesh = plsc.VectorSubcoreMesh(core_axis_name="c", subcore_axis_name="s")


def _t_in_kernel(wvt_ref, out_ref):
    v = wvt_ref[...]
    z = jnp.zeros((PAD_DIM - EMBED_DIM, VB), jnp.float32)
    out_ref[...] = jnp.concatenate([v, z], axis=0).T


_transpose_in = pl.pallas_call(
    _t_in_kernel,
    grid=(NVB,),
    in_specs=[pl.BlockSpec((EMBED_DIM, VB), lambda i: (0, i))],
    out_specs=pl.BlockSpec((VB, PAD_DIM), lambda i: (i, 0)),
    out_shape=jax.ShapeDtypeStruct((NVB * VB, PAD_DIM), jnp.float32),
)


@functools.partial(
    pl.kernel,
    mesh=_mesh,
    out_type=jax.ShapeDtypeStruct((HALF, SENT_PAD, PAD_DIM), jnp.float32),
    scratch_types=[
        pltpu.VMEM((ROWS_PER_WORKER, SENT_PAD), jnp.int32),
        pltpu.VMEM((4, SENT_PAD, PAD_DIM), jnp.float32),
        pltpu.SemaphoreType.DMA,
        pltpu.SemaphoreType.DMA,
    ],
)
def _gather_kernel(idx_hbm, table_hbm, out_hbm, idx_v, rows_v, sem_g, sem_w):
    wid = lax.axis_index("s") * NUM_CORES + lax.axis_index("c")
    base = wid * ROWS_PER_WORKER
    pltpu.sync_copy(idx_hbm.at[wid], idx_v)

    pltpu.async_copy(table_hbm.at[idx_v.at[0]], rows_v.at[0], sem_g)
    pltpu.async_copy(table_hbm.at[idx_v.at[1]], rows_v.at[1], sem_g)

    def body(b, carry):
        pltpu.make_async_copy(out_hbm.at[0], rows_v.at[0], sem_g).wait()

        @pl.when(b >= 2)
        def _():
            pltpu.make_async_copy(out_hbm.at[0], rows_v.at[0], sem_w).wait()

        @pl.when(b + 2 < ROWS_PER_WORKER)
        def _():
            nxt = lax.rem(b + 2, 4)
            pltpu.async_copy(table_hbm.at[idx_v.at[b + 2]], rows_v.at[nxt],
                             sem_g)

        cur = lax.rem(b, 4)
        pltpu.async_copy(rows_v.at[cur], out_hbm.at[base + b], sem_w)
        return carry

    lax.fori_loop(0, ROWS_PER_WORKER, body, 0, unroll=False)
    pltpu.make_async_copy(out_hbm.at[0], rows_v.at[0], sem_w).wait()
    pltpu.make_async_copy(out_hbm.at[0], rows_v.at[0], sem_w).wait()


def _t_out_a(rows_ref, out_ref):
    for j in range(SENT_LEN):
        out_ref[j] = rows_ref[:, j, :].T[:EMBED_DIM]


def _t_out_b(rows_ref, base_ref, out_ref):
    del base_ref
    for j in range(SENT_LEN):
        out_ref[j] = rows_ref[:, j, :].T[:EMBED_DIM]


_transpose_out_a = pl.pallas_call(
    _t_out_a,
    grid=(HALF // IB,),
    in_specs=[pl.BlockSpec((IB, SENT_PAD, PAD_DIM), lambda i: (i, 0, 0))],
    out_specs=pl.BlockSpec((SENT_LEN, EMBED_DIM, IB), lambda i: (0, 0, i)),
    out_shape=jax.ShapeDtypeStruct((SENT_LEN, EMBED_DIM, BATCH), jnp.float32),
)

_transpose_out_b = pl.pallas_call(
    _t_out_b,
    grid=(HALF // IB,),
    in_specs=[
        pl.BlockSpec((IB, SENT_PAD, PAD_DIM), lambda i: (i, 0, 0)),
        pl.BlockSpec(memory_space=pl.ANY),
    ],
    out_specs=pl.BlockSpec((SENT_LEN, EMBED_DIM, IB),
                           lambda i: (0, 0, i + HALF // IB)),
    out_shape=jax.ShapeDtypeStruct((SENT_LEN, EMBED_DIM, BATCH), jnp.float32),
    input_output_aliases={1: 0},
)


def kernel(x, word_vectors):
    wvt = jnp.swapaxes(word_vectors, 0, 1)
    table = _transpose_in(wvt)
    idx3 = x.reshape(2, NUM_WORKERS, ROWS_PER_WORKER, SENT_LEN).astype(
        jnp.int32)
    fill = (
        jnp.arange(SENT_PAD - SENT_LEN, dtype=jnp.int32)[None, None, None, :]
        + 8 * jnp.arange(ROWS_PER_WORKER, dtype=jnp.int32)[None, None, :, None]
        + 128 * jnp.arange(NUM_WORKERS, dtype=jnp.int32)[None, :, None, None]
        + 4096 * jnp.arange(2, dtype=jnp.int32)[:, None, None, None]
    )
    idxp = jnp.concatenate([idx3, fill], axis=3)
    rows_a = _gather_kernel(idxp[0], table)
    rows_b = _gather_kernel(idxp[1], table)
    g = _transpose_out_a(rows_a)
    g = _transpose_out_b(rows_b, g)
    return jnp.transpose(g, (2, 0, 1))
